# Initial kernel scaffold; baseline (speedup 1.0000x reference)
import jax
import jax.numpy as jnp
from jax.experimental import pallas as pl


def kernel(x, Wi, bi, Wf, bf, Wo, bo, Wc, bc, Wg, bg):
    raise NotImplementedError("write your pallas kernel here")


# speedup vs baseline: 9.8685x; 9.8685x over previous
import numpy as np
import jax
import jax.numpy as jnp
from jax.experimental import pallas as pl

SEQ_LEN = 16
HIDDEN = 32
INPUT = 256
BATCH = 1024


def _build_adj():
    adj_list = [[0, 2, 5, 8, 11], [0, 1, 4, 7, 10], [0, 3, 6, 9, 12, 15],
                [9, 14, 17, 19, 21], [9, 13, 16, 18, 20]]
    num_nodes = max(max(sub) for sub in adj_list) + 1
    adj = np.zeros((num_nodes, num_nodes), dtype=np.float32)
    for sub in adj_list:
        for i in range(len(sub)):
            for j in range(i + 1, len(sub)):
                adj[sub[i], sub[j]] = 1.0
                adj[sub[j], sub[i]] = 1.0
    deg = adj.sum(axis=0)
    norm = 1.0 / np.sqrt(np.clip(deg, 1.0, None))
    a_hat = norm[:, None] * adj * norm[None, :]
    return a_hat.astype(np.float32), num_nodes


A_HAT, NUM_NODES = _build_adj()
NH = NUM_NODES * HIDDEN

A_EXP = np.kron(A_HAT.T, np.ones((HIDDEN, HIDDEN), np.float32))
FT = np.tile(np.eye(HIDDEN, dtype=np.float32), (1, NUM_NODES))
F = FT.T.copy()

BT = 128


def _kern(x_ref, wi_ref, bi_ref, wf_ref, bf_ref, wo_ref, bo_ref,
          wc_ref, bc_ref, wg_ref, bgt_ref, aexp_ref, f_ref, ft_ref,
          out_ref):
    f32 = jnp.float32
    x = x_ref[...]
    ft = ft_ref[...]
    bgt = bgt_ref[...]

    def gate_offset(w_ref, b_ref):
        g = jnp.dot(x, w_ref[...], preferred_element_type=f32) + b_ref[...]
        return jnp.dot(g, ft, preferred_element_type=f32) + bgt

    oi = gate_offset(wi_ref, bi_ref)
    of = gate_offset(wf_ref, bf_ref)
    oo = gate_offset(wo_ref, bo_ref)
    oc = gate_offset(wc_ref, bc_ref)

    w_exp = jnp.dot(jnp.dot(f_ref[...], wg_ref[...], preferred_element_type=f32),
                    ft, preferred_element_type=f32)
    m = aexp_ref[...] * w_exp

    h = None
    c = None
    for t in range(SEQ_LEN):
        if t == 0:
            gi, gf, go, gc = oi, of, oo, oc
        else:
            g = jnp.dot(h, m, preferred_element_type=f32)
            gi, gf, go, gc = oi + g, of + g, oo + g, oc + g
        i_t = jax.nn.sigmoid(gi)
        f_t = jax.nn.sigmoid(gf)
        o_t = jax.nn.sigmoid(go)
        c_t = jnp.tanh(gc)
        c = i_t * c_t if c is None else f_t * c + i_t * c_t
        h = o_t * jnp.tanh(c)
        out_ref[:, t * NH:(t + 1) * NH] = h


def kernel(x, Wi, bi, Wf, bf, Wo, bo, Wc, bc, Wg, bg):
    B = x.shape[0]
    grid = (B // BT,)
    full = lambda i: (0, 0)
    tile = lambda i: (i, 0)
    bgt = jnp.tile(bg, NUM_NODES)[None, :]
    args = (
        x, Wi, bi[None, :], Wf, bf[None, :], Wo, bo[None, :],
        Wc, bc[None, :], Wg, bgt,
        jnp.asarray(A_EXP), jnp.asarray(F), jnp.asarray(FT),
    )
    in_specs = [
        pl.BlockSpec((BT, INPUT), tile),
        pl.BlockSpec((INPUT, HIDDEN), full), pl.BlockSpec((1, HIDDEN), full),
        pl.BlockSpec((INPUT, HIDDEN), full), pl.BlockSpec((1, HIDDEN), full),
        pl.BlockSpec((INPUT, HIDDEN), full), pl.BlockSpec((1, HIDDEN), full),
        pl.BlockSpec((INPUT, HIDDEN), full), pl.BlockSpec((1, HIDDEN), full),
        pl.BlockSpec((HIDDEN, HIDDEN), full),
        pl.BlockSpec((1, NH), full),
        pl.BlockSpec((NH, NH), full),
        pl.BlockSpec((NH, HIDDEN), full),
        pl.BlockSpec((HIDDEN, NH), full),
    ]
    out = pl.pallas_call(
        _kern,
        grid=grid,
        in_specs=in_specs,
        out_specs=pl.BlockSpec((BT, SEQ_LEN * NH), tile),
        out_shape=jax.ShapeDtypeStruct((B, SEQ_LEN * NH), jnp.float32),
    )(*args)
    return out


# speedup vs baseline: 12.1763x; 1.2339x over previous
import numpy as np
import jax
import jax.numpy as jnp
from jax.experimental import pallas as pl

SEQ_LEN = 16
HIDDEN = 32
INPUT = 256
BATCH = 1024


def _build_adj():
    adj_list = [[0, 2, 5, 8, 11], [0, 1, 4, 7, 10], [0, 3, 6, 9, 12, 15],
                [9, 14, 17, 19, 21], [9, 13, 16, 18, 20]]
    num_nodes = max(max(sub) for sub in adj_list) + 1
    adj = np.zeros((num_nodes, num_nodes), dtype=np.float32)
    for sub in adj_list:
        for i in range(len(sub)):
            for j in range(i + 1, len(sub)):
                adj[sub[i], sub[j]] = 1.0
                adj[sub[j], sub[i]] = 1.0
    deg = adj.sum(axis=0)
    norm = 1.0 / np.sqrt(np.clip(deg, 1.0, None))
    a_hat = norm[:, None] * adj * norm[None, :]
    return a_hat.astype(np.float32), num_nodes


A_HAT, NUM_NODES = _build_adj()
NH = NUM_NODES * HIDDEN

A_EXP = np.kron(A_HAT.T, np.ones((HIDDEN, HIDDEN), np.float32))
FT = np.tile(np.eye(HIDDEN, dtype=np.float32), (1, NUM_NODES))
F = FT.T.copy()

BT = 128


def _kern(x_ref, wi_ref, bi_ref, wf_ref, bf_ref, wo_ref, bo_ref,
          wc_ref, bc_ref, wg_ref, bgt_ref, aexp_ref, f_ref, ft_ref,
          out_ref):
    f32 = jnp.float32
    x = x_ref[...]
    ft = ft_ref[...]
    bgt = bgt_ref[...]

    def gate_offset(w_ref, b_ref):
        g = jnp.dot(x, w_ref[...], preferred_element_type=f32) + b_ref[...]
        return jnp.dot(g, ft, preferred_element_type=f32) + bgt

    oi = gate_offset(wi_ref, bi_ref)
    of = gate_offset(wf_ref, bf_ref)
    oo = gate_offset(wo_ref, bo_ref)
    oc = gate_offset(wc_ref, bc_ref)

    w_exp = jnp.dot(jnp.dot(f_ref[...], wg_ref[...], preferred_element_type=f32),
                    ft, preferred_element_type=f32)
    m = aexp_ref[...] * w_exp

    h = None
    c = None
    for t in range(SEQ_LEN):
        if t == 0:
            gi, gf, go, gc = oi, of, oo, oc
        else:
            g = jnp.dot(h, m, preferred_element_type=f32)
            gi, gf, go, gc = oi + g, of + g, oo + g, oc + g
        i_t = 0.5 + 0.5 * jnp.tanh(0.5 * gi)
        f_t = 0.5 + 0.5 * jnp.tanh(0.5 * gf)
        o_t = 0.5 + 0.5 * jnp.tanh(0.5 * go)
        c_t = jnp.tanh(gc)
        c = i_t * c_t if c is None else f_t * c + i_t * c_t
        h = o_t * jnp.tanh(c)
        out_ref[:, t * NH:(t + 1) * NH] = h


def kernel(x, Wi, bi, Wf, bf, Wo, bo, Wc, bc, Wg, bg):
    B = x.shape[0]
    grid = (B // BT,)
    full = lambda i: (0, 0)
    tile = lambda i: (i, 0)
    bgt = jnp.tile(bg, NUM_NODES)[None, :]
    args = (
        x, Wi, bi[None, :], Wf, bf[None, :], Wo, bo[None, :],
        Wc, bc[None, :], Wg, bgt,
        jnp.asarray(A_EXP), jnp.asarray(F), jnp.asarray(FT),
    )
    in_specs = [
        pl.BlockSpec((BT, INPUT), tile),
        pl.BlockSpec((INPUT, HIDDEN), full), pl.BlockSpec((1, HIDDEN), full),
        pl.BlockSpec((INPUT, HIDDEN), full), pl.BlockSpec((1, HIDDEN), full),
        pl.BlockSpec((INPUT, HIDDEN), full), pl.BlockSpec((1, HIDDEN), full),
        pl.BlockSpec((INPUT, HIDDEN), full), pl.BlockSpec((1, HIDDEN), full),
        pl.BlockSpec((HIDDEN, HIDDEN), full),
        pl.BlockSpec((1, NH), full),
        pl.BlockSpec((NH, NH), full),
        pl.BlockSpec((NH, HIDDEN), full),
        pl.BlockSpec((HIDDEN, NH), full),
    ]
    out = pl.pallas_call(
        _kern,
        grid=grid,
        in_specs=in_specs,
        out_specs=pl.BlockSpec((BT, SEQ_LEN * NH), tile),
        out_shape=jax.ShapeDtypeStruct((B, SEQ_LEN * NH), jnp.float32),
    )(*args)
    return out


# speedup vs baseline: 14.8365x; 1.2185x over previous
import numpy as np
import jax
import jax.numpy as jnp
from jax.experimental import pallas as pl

SEQ_LEN = 16
HIDDEN = 32
INPUT = 256
BATCH = 1024


def _build_adj():
    adj_list = [[0, 2, 5, 8, 11], [0, 1, 4, 7, 10], [0, 3, 6, 9, 12, 15],
                [9, 14, 17, 19, 21], [9, 13, 16, 18, 20]]
    num_nodes = max(max(sub) for sub in adj_list) + 1
    adj = np.zeros((num_nodes, num_nodes), dtype=np.float32)
    for sub in adj_list:
        for i in range(len(sub)):
            for j in range(i + 1, len(sub)):
                adj[sub[i], sub[j]] = 1.0
                adj[sub[j], sub[i]] = 1.0
    deg = adj.sum(axis=0)
    norm = 1.0 / np.sqrt(np.clip(deg, 1.0, None))
    a_hat = norm[:, None] * adj * norm[None, :]
    return a_hat.astype(np.float32), num_nodes


A_HAT, NUM_NODES = _build_adj()
NH = NUM_NODES * HIDDEN

A_EXP = np.kron(A_HAT.T, np.ones((HIDDEN, HIDDEN), np.float32))
FT = np.tile(np.eye(HIDDEN, dtype=np.float32), (1, NUM_NODES))
F = FT.T.copy()

BT = 256


def _kern(x_ref, wi_ref, bi_ref, wf_ref, bf_ref, wo_ref, bo_ref,
          wc_ref, bc_ref, wg_ref, bgt_ref, aexp_ref, f_ref, ft_ref,
          out_ref):
    f32 = jnp.float32
    x = x_ref[...]
    ft = ft_ref[...]
    bgt = bgt_ref[...]

    def gate_offset(w_ref, b_ref):
        g = jnp.dot(x, w_ref[...], preferred_element_type=f32) + b_ref[...]
        return jnp.dot(g, ft, preferred_element_type=f32) + bgt

    oi = gate_offset(wi_ref, bi_ref)
    of = gate_offset(wf_ref, bf_ref)
    oo = gate_offset(wo_ref, bo_ref)
    oc = gate_offset(wc_ref, bc_ref)

    w_exp = jnp.dot(jnp.dot(f_ref[...], wg_ref[...], preferred_element_type=f32),
                    ft, preferred_element_type=f32)
    m = aexp_ref[...] * w_exp

    h = None
    c = None
    for t in range(SEQ_LEN):
        if t == 0:
            gi, gf, go, gc = oi, of, oo, oc
        else:
            g = jnp.dot(h, m, preferred_element_type=f32)
            gi, gf, go, gc = oi + g, of + g, oo + g, oc + g
        i_t = 0.5 + 0.5 * jnp.tanh(0.5 * gi)
        f_t = 0.5 + 0.5 * jnp.tanh(0.5 * gf)
        o_t = 0.5 + 0.5 * jnp.tanh(0.5 * go)
        c_t = jnp.tanh(gc)
        c = i_t * c_t if c is None else f_t * c + i_t * c_t
        h = o_t * jnp.tanh(c)
        out_ref[:, t * NH:(t + 1) * NH] = h


def kernel(x, Wi, bi, Wf, bf, Wo, bo, Wc, bc, Wg, bg):
    B = x.shape[0]
    grid = (B // BT,)
    full = lambda i: (0, 0)
    tile = lambda i: (i, 0)
    bgt = jnp.tile(bg, NUM_NODES)[None, :]
    args = (
        x, Wi, bi[None, :], Wf, bf[None, :], Wo, bo[None, :],
        Wc, bc[None, :], Wg, bgt,
        jnp.asarray(A_EXP), jnp.asarray(F), jnp.asarray(FT),
    )
    in_specs = [
        pl.BlockSpec((BT, INPUT), tile),
        pl.BlockSpec((INPUT, HIDDEN), full), pl.BlockSpec((1, HIDDEN), full),
        pl.BlockSpec((INPUT, HIDDEN), full), pl.BlockSpec((1, HIDDEN), full),
        pl.BlockSpec((INPUT, HIDDEN), full), pl.BlockSpec((1, HIDDEN), full),
        pl.BlockSpec((INPUT, HIDDEN), full), pl.BlockSpec((1, HIDDEN), full),
        pl.BlockSpec((HIDDEN, HIDDEN), full),
        pl.BlockSpec((1, NH), full),
        pl.BlockSpec((NH, NH), full),
        pl.BlockSpec((NH, HIDDEN), full),
        pl.BlockSpec((HIDDEN, NH), full),
    ]
    out = pl.pallas_call(
        _kern,
        grid=grid,
        in_specs=in_specs,
        out_specs=pl.BlockSpec((BT, SEQ_LEN * NH), tile),
        out_shape=jax.ShapeDtypeStruct((B, SEQ_LEN * NH), jnp.float32),
    )(*args)
    return out
